# baseline (speedup 1.0000x reference)
import functools

import jax
import jax.numpy as jnp
from jax import lax
from jax.experimental import pallas as pl
from jax.experimental.pallas import tpu as pltpu
from jax.experimental.pallas import tpu_sc as plsc

BATCH = 16384
EMBED = 64
HID = 32

NC = 2
NS = 16
LANES = 16
NW = NC * NS
ROWS_PER_W = BATCH // NW
KCH = 256
NCH = ROWS_PER_W // KCH

CB2 = 16384
CB2_SHIFT = 14


def _tc_pack_transpose(tabT):
    n = tabT.shape[1]
    nblk = -(-n // (2 * CB2))

    def body(a_ref, b_ref, out_ref):
        ap = pltpu.bitcast(a_ref[...].T.astype(jnp.bfloat16), jnp.float32)
        bp = pltpu.bitcast(b_ref[...].T.astype(jnp.bfloat16), jnp.float32)
        out_ref[...] = jnp.concatenate([ap, bp], axis=1)

    HB = CB2 // 2
    last = (n - 1) // HB
    return pl.pallas_call(
        body,
        grid=(nblk, 2),
        in_specs=[
            pl.BlockSpec((EMBED, HB),
                         lambda i, j, last=last:
                         (0, jnp.minimum(4 * i + j, last))),
            pl.BlockSpec((EMBED, HB),
                         lambda i, j, last=last:
                         (0, jnp.minimum(4 * i + 2 + j, last))),
        ],
        out_specs=pl.BlockSpec((HB // 2, 2 * EMBED), lambda i, j: (2 * i + j, 0)),
        out_shape=jax.ShapeDtypeStruct((nblk * CB2 // 2, 2 * EMBED),
                                       jnp.float32),
    )(tabT, tabT)


def _packed_row(v):
    chunk = v >> CB2_SHIFT
    return ((chunk >> 1) << (CB2_SHIFT - 1)) + ((v & (CB2 - 1)) >> 1)


def _sc_gather(pk, ids):
    mesh = plsc.VectorSubcoreMesh(core_axis_name="c", subcore_axis_name="s")

    @functools.partial(
        pl.kernel,
        mesh=mesh,
        out_type=jax.ShapeDtypeStruct((BATCH, 2 * EMBED), jnp.float32),
        scratch_types=[
            pltpu.VMEM((ROWS_PER_W,), jnp.int32),
            pltpu.VMEM((KCH, 2 * EMBED), jnp.float32),
            pltpu.VMEM((KCH, 2 * EMBED), jnp.float32),
            pltpu.SemaphoreType.DMA,
            pltpu.SemaphoreType.DMA,
        ],
    )
    def gather_kernel(pk_hbm, id_hbm, out_hbm, idx_v, buf0, buf1, sem0, sem1):
        wid = lax.axis_index("s") * NC + lax.axis_index("c")
        base = wid * ROWS_PER_W
        pltpu.sync_copy(id_hbm.at[pl.ds(base, ROWS_PER_W)], idx_v)
        bufs = (buf0, buf1)
        sems = (sem0, sem1)

        def fire(c):
            def body(j, carry, c=c):
                off = c * KCH + j * LANES
                vec = _packed_row(idx_v[pl.ds(off, LANES)])
                for k in range(LANES):
                    pltpu.async_copy(pk_hbm.at[vec[k]],
                                     bufs[c % 2].at[j * LANES + k],
                                     sems[c % 2])
                return carry

            lax.fori_loop(0, KCH // LANES, body, 0)

        def drain_store(c):
            pltpu.make_async_copy(out_hbm.at[pl.ds(0, KCH)], bufs[c % 2],
                                  sems[c % 2]).wait()
            pltpu.sync_copy(bufs[c % 2], out_hbm.at[pl.ds(base + c * KCH, KCH)])

        fire(0)
        for c in range(NCH):
            if c + 1 < NCH:
                fire(c + 1)
            drain_store(c)

    return gather_kernel(pk, ids)


def _mlp_body(gu_ref, ga_ref, uid_ref, aid_ref,
              w1u_ref, b1u_ref, w2u_ref, b2u_ref,
              w1a_ref, b1a_ref, w2a_ref, b2a_ref, out_ref):
    def unpack(g, ids):
        p = ((ids >> CB2_SHIFT) & 1) == 1
        w = jnp.where(p, g[:, EMBED:], g[:, :EMBED])
        wi = lax.bitcast_convert_type(w, jnp.int32)
        lo = lax.bitcast_convert_type(wi << 16, jnp.float32)
        hi = lax.bitcast_convert_type(wi & jnp.int32(-65536), jnp.float32)
        q = (ids & 1) == 1
        return jnp.where(q, hi, lo)

    eu = unpack(gu_ref[...], uid_ref[...])
    ea = unpack(ga_ref[...], aid_ref[...])
    u = jnp.dot(eu, w1u_ref[...], preferred_element_type=jnp.float32)
    u = jnp.maximum(u + b1u_ref[...], 0.0)
    u = jnp.dot(u, w2u_ref[...], preferred_element_type=jnp.float32)
    u = jnp.maximum(u + b2u_ref[...], 0.0)
    a = jnp.dot(ea, w1a_ref[...], preferred_element_type=jnp.float32)
    a = jnp.maximum(a + b1a_ref[...], 0.0)
    a = jnp.dot(a, w2a_ref[...], preferred_element_type=jnp.float32)
    a = jnp.maximum(a + b2a_ref[...], 0.0)
    out_ref[...] = jnp.sum(u * a, axis=1)


def _tc_mlp(gu, ga, uids, aids, W1u, b1u, W2u, b2u, W1a, b1a, W2a, b2a):
    BLK = 4096
    grid = BATCH // BLK
    espec = pl.BlockSpec((BLK, 2 * EMBED), lambda i: (i, 0))
    ispec = pl.BlockSpec((BLK, 1), lambda i: (i, 0))
    wspec = pl.BlockSpec((EMBED, HID), lambda i: (0, 0))
    w2spec = pl.BlockSpec((HID, HID), lambda i: (0, 0))
    bspec = pl.BlockSpec((1, HID), lambda i: (0, 0))
    return pl.pallas_call(
        _mlp_body,
        grid=(grid,),
        in_specs=[espec, espec, ispec, ispec,
                  wspec, bspec, w2spec, bspec,
                  wspec, bspec, w2spec, bspec],
        out_specs=pl.BlockSpec((BLK,), lambda i: (i,)),
        out_shape=jax.ShapeDtypeStruct((BATCH,), jnp.float32),
    )(gu, ga, uids.reshape(BATCH, 1), aids.reshape(BATCH, 1),
      W1u.T, b1u.reshape(1, HID), W2u.T, b2u.reshape(1, HID),
      W1a.T, b1a.reshape(1, HID), W2a.T, b2a.reshape(1, HID))


def kernel(user_ids, anime_ids, user_table, anime_table,
           W1u, b1u, W2u, b2u, W1a, b1a, W2a, b2a):
    uids = user_ids.astype(jnp.int32)
    aids = anime_ids.astype(jnp.int32)
    upk = _tc_pack_transpose(user_table.T)
    gu = _sc_gather(upk, uids)
    apk = _tc_pack_transpose(anime_table.T)
    ga = _sc_gather(apk, aids)
    return _tc_mlp(gu, ga, uids, aids,
                   W1u, b1u, W2u, b2u, W1a, b1a, W2a, b2a)


# speedup vs baseline: 1.0807x; 1.0807x over previous
import functools

import jax
import jax.numpy as jnp
from jax import lax
from jax.experimental import pallas as pl
from jax.experimental.pallas import tpu as pltpu
from jax.experimental.pallas import tpu_sc as plsc

BATCH = 16384
EMBED = 64
HID = 32

NC = 2
NS = 16
LANES = 16
NW = NC * NS
ROWS_PER_W = BATCH // NW
KCH = 256
NCH = ROWS_PER_W // KCH

CB2 = 16384
CB2_SHIFT = 14


def _tc_pack_transpose(tabT):
    n = tabT.shape[1]
    nblk = -(-n // (2 * CB2))

    def body(a_ref, b_ref, out_ref):
        ap = pltpu.bitcast(a_ref[...].T.astype(jnp.bfloat16), jnp.float32)
        bp = pltpu.bitcast(b_ref[...].T.astype(jnp.bfloat16), jnp.float32)
        out_ref[...] = jnp.concatenate([ap, bp], axis=1)

    last = (n - 1) // CB2
    return pl.pallas_call(
        body,
        grid=(nblk,),
        in_specs=[
            pl.BlockSpec((EMBED, CB2), lambda i: (0, 2 * i)),
            pl.BlockSpec((EMBED, CB2),
                         lambda i, last=last: (0, jnp.minimum(2 * i + 1, last))),
        ],
        out_specs=pl.BlockSpec((CB2 // 2, 2 * EMBED), lambda i: (i, 0)),
        out_shape=jax.ShapeDtypeStruct((nblk * CB2 // 2, 2 * EMBED),
                                       jnp.float32),
    )(tabT, tabT)


def _packed_row(v):
    chunk = v >> CB2_SHIFT
    return ((chunk >> 1) << (CB2_SHIFT - 1)) + ((v & (CB2 - 1)) >> 1)


def _sc_gather(pk, ids):
    mesh = plsc.VectorSubcoreMesh(core_axis_name="c", subcore_axis_name="s")

    @functools.partial(
        pl.kernel,
        mesh=mesh,
        out_type=jax.ShapeDtypeStruct((BATCH, 2 * EMBED), jnp.float32),
        scratch_types=[
            pltpu.VMEM((ROWS_PER_W,), jnp.int32),
            pltpu.VMEM((KCH, 2 * EMBED), jnp.float32),
            pltpu.VMEM((KCH, 2 * EMBED), jnp.float32),
            pltpu.SemaphoreType.DMA,
            pltpu.SemaphoreType.DMA,
        ],
    )
    def gather_kernel(pk_hbm, id_hbm, out_hbm, idx_v, buf0, buf1, sem0, sem1):
        wid = lax.axis_index("s") * NC + lax.axis_index("c")
        base = wid * ROWS_PER_W
        pltpu.sync_copy(id_hbm.at[pl.ds(base, ROWS_PER_W)], idx_v)
        bufs = (buf0, buf1)
        sems = (sem0, sem1)

        def fire(c):
            def body(j, carry, c=c):
                off = c * KCH + j * LANES
                vec = _packed_row(idx_v[pl.ds(off, LANES)])
                for k in range(LANES):
                    pltpu.async_copy(pk_hbm.at[vec[k]],
                                     bufs[c % 2].at[j * LANES + k],
                                     sems[c % 2])
                return carry

            lax.fori_loop(0, KCH // LANES, body, 0)

        def drain_store(c):
            pltpu.make_async_copy(out_hbm.at[pl.ds(0, KCH)], bufs[c % 2],
                                  sems[c % 2]).wait()
            pltpu.sync_copy(bufs[c % 2], out_hbm.at[pl.ds(base + c * KCH, KCH)])

        fire(0)
        for c in range(NCH):
            if c + 1 < NCH:
                fire(c + 1)
            drain_store(c)

    return gather_kernel(pk, ids)


def _mlp_body(gu_ref, ga_ref, uid_ref, aid_ref,
              w1u_ref, b1u_ref, w2u_ref, b2u_ref,
              w1a_ref, b1a_ref, w2a_ref, b2a_ref, out_ref):
    def unpack(g, ids):
        p = ((ids >> CB2_SHIFT) & 1) == 1
        w = jnp.where(p, g[:, EMBED:], g[:, :EMBED])
        wi = lax.bitcast_convert_type(w, jnp.int32)
        lo = lax.bitcast_convert_type(wi << 16, jnp.float32)
        hi = lax.bitcast_convert_type(wi & jnp.int32(-65536), jnp.float32)
        q = (ids & 1) == 1
        return jnp.where(q, hi, lo)

    eu = unpack(gu_ref[...], uid_ref[...])
    ea = unpack(ga_ref[...], aid_ref[...])
    u = jnp.dot(eu, w1u_ref[...], preferred_element_type=jnp.float32)
    u = jnp.maximum(u + b1u_ref[...], 0.0)
    u = jnp.dot(u, w2u_ref[...], preferred_element_type=jnp.float32)
    u = jnp.maximum(u + b2u_ref[...], 0.0)
    a = jnp.dot(ea, w1a_ref[...], preferred_element_type=jnp.float32)
    a = jnp.maximum(a + b1a_ref[...], 0.0)
    a = jnp.dot(a, w2a_ref[...], preferred_element_type=jnp.float32)
    a = jnp.maximum(a + b2a_ref[...], 0.0)
    out_ref[...] = jnp.sum(u * a, axis=1)


def _tc_mlp(gu, ga, uids, aids, W1u, b1u, W2u, b2u, W1a, b1a, W2a, b2a):
    BLK = 4096
    grid = BATCH // BLK
    espec = pl.BlockSpec((BLK, 2 * EMBED), lambda i: (i, 0))
    ispec = pl.BlockSpec((BLK, 1), lambda i: (i, 0))
    wspec = pl.BlockSpec((EMBED, HID), lambda i: (0, 0))
    w2spec = pl.BlockSpec((HID, HID), lambda i: (0, 0))
    bspec = pl.BlockSpec((1, HID), lambda i: (0, 0))
    return pl.pallas_call(
        _mlp_body,
        grid=(grid,),
        in_specs=[espec, espec, ispec, ispec,
                  wspec, bspec, w2spec, bspec,
                  wspec, bspec, w2spec, bspec],
        out_specs=pl.BlockSpec((BLK,), lambda i: (i,)),
        out_shape=jax.ShapeDtypeStruct((BATCH,), jnp.float32),
    )(gu, ga, uids.reshape(BATCH, 1), aids.reshape(BATCH, 1),
      W1u.T, b1u.reshape(1, HID), W2u.T, b2u.reshape(1, HID),
      W1a.T, b1a.reshape(1, HID), W2a.T, b2a.reshape(1, HID))


def kernel(user_ids, anime_ids, user_table, anime_table,
           W1u, b1u, W2u, b2u, W1a, b1a, W2a, b2a):
    uids = user_ids.astype(jnp.int32)
    aids = anime_ids.astype(jnp.int32)
    upk = _tc_pack_transpose(user_table.T)
    gu = _sc_gather(upk, uids)
    apk = _tc_pack_transpose(anime_table.T)
    ga = _sc_gather(apk, aids)
    return _tc_mlp(gu, ga, uids, aids,
                   W1u, b1u, W2u, b2u, W1a, b1a, W2a, b2a)


# speedup vs baseline: 1.0833x; 1.0024x over previous
import functools

import jax
import jax.numpy as jnp
from jax import lax
from jax.experimental import pallas as pl
from jax.experimental.pallas import tpu as pltpu
from jax.experimental.pallas import tpu_sc as plsc

BATCH = 16384
EMBED = 64
HID = 32

NC = 2
NS = 16
LANES = 16
NW = NC * NS
ROWS_PER_W = BATCH // NW
KCH = 256
NCH = ROWS_PER_W // KCH

CB2 = 16384
CB2_SHIFT = 14


def _tc_pack_transpose(tabT):
    n = tabT.shape[1]
    nblk = -(-n // (2 * CB2))

    def body(a_ref, b_ref, out_ref):
        ap = pltpu.bitcast(a_ref[...].T.astype(jnp.bfloat16), jnp.float32)
        bp = pltpu.bitcast(b_ref[...].T.astype(jnp.bfloat16), jnp.float32)
        out_ref[...] = jnp.concatenate([ap, bp], axis=1)

    last = (n - 1) // CB2
    return pl.pallas_call(
        body,
        grid=(nblk,),
        in_specs=[
            pl.BlockSpec((EMBED, CB2), lambda i: (0, 2 * i)),
            pl.BlockSpec((EMBED, CB2),
                         lambda i, last=last: (0, jnp.minimum(2 * i + 1, last))),
        ],
        out_specs=pl.BlockSpec((CB2 // 2, 2 * EMBED), lambda i: (i, 0)),
        out_shape=jax.ShapeDtypeStruct((nblk * CB2 // 2, 2 * EMBED),
                                       jnp.float32),
    )(tabT, tabT)


def _packed_row(v):
    chunk = v >> CB2_SHIFT
    return ((chunk >> 1) << (CB2_SHIFT - 1)) + ((v & (CB2 - 1)) >> 1)


def _sc_gather(pk, ids):
    mesh = plsc.VectorSubcoreMesh(core_axis_name="c", subcore_axis_name="s")

    @functools.partial(
        pl.kernel,
        mesh=mesh,
        out_type=jax.ShapeDtypeStruct((BATCH, 2 * EMBED), jnp.float32),
        scratch_types=[
            pltpu.VMEM((ROWS_PER_W,), jnp.int32),
            pltpu.VMEM((KCH, 2 * EMBED), jnp.float32),
            pltpu.VMEM((KCH, 2 * EMBED), jnp.float32),
            pltpu.SemaphoreType.DMA,
            pltpu.SemaphoreType.DMA,
        ],
    )
    def gather_kernel(pk_hbm, id_hbm, out_hbm, idx_v, buf0, buf1, sem0, sem1):
        wid = lax.axis_index("s") * NC + lax.axis_index("c")
        base = wid * ROWS_PER_W
        pltpu.sync_copy(id_hbm.at[pl.ds(base, ROWS_PER_W)], idx_v)
        bufs = (buf0, buf1)
        sems = (sem0, sem1)

        def fire(c):
            def body(j, carry, c=c):
                off = c * KCH + j * LANES
                vec = _packed_row(idx_v[pl.ds(off, LANES)])
                for k in range(LANES):
                    pltpu.async_copy(pk_hbm.at[vec[k]],
                                     bufs[c % 2].at[j * LANES + k],
                                     sems[c % 2])
                return carry

            lax.fori_loop(0, KCH // LANES, body, 0)

        def drain_store(c):
            pltpu.make_async_copy(out_hbm.at[pl.ds(0, KCH)], bufs[c % 2],
                                  sems[c % 2]).wait()
            pltpu.sync_copy(bufs[c % 2], out_hbm.at[pl.ds(base + c * KCH, KCH)])

        fire(0)
        for c in range(NCH):
            if c + 1 < NCH:
                fire(c + 1)
            drain_store(c)

    return gather_kernel(pk, ids)


def _mlp_body(gu_ref, ga_ref, uid_ref, aid_ref,
              w1u_ref, b1u_ref, w2u_ref, b2u_ref,
              w1a_ref, b1a_ref, w2a_ref, b2a_ref, out_ref):
    def unpack(g, ids):
        p = ((ids >> CB2_SHIFT) & 1) == 1
        w = jnp.where(p, g[:, EMBED:], g[:, :EMBED])
        wi = lax.bitcast_convert_type(w, jnp.int32)
        lo = lax.bitcast_convert_type(wi << 16, jnp.float32)
        hi = lax.bitcast_convert_type(wi & jnp.int32(-65536), jnp.float32)
        q = (ids & 1) == 1
        return jnp.where(q, hi, lo)

    eu = unpack(gu_ref[...], uid_ref[...])
    ea = unpack(ga_ref[...], aid_ref[...])
    u = jnp.dot(eu, w1u_ref[...], preferred_element_type=jnp.float32)
    u = jnp.maximum(u + b1u_ref[...], 0.0)
    u = jnp.dot(u, w2u_ref[...], preferred_element_type=jnp.float32)
    u = jnp.maximum(u + b2u_ref[...], 0.0)
    a = jnp.dot(ea, w1a_ref[...], preferred_element_type=jnp.float32)
    a = jnp.maximum(a + b1a_ref[...], 0.0)
    a = jnp.dot(a, w2a_ref[...], preferred_element_type=jnp.float32)
    a = jnp.maximum(a + b2a_ref[...], 0.0)
    out_ref[...] = jnp.sum(u * a, axis=1)


def _tc_mlp(gu, ga, uids, aids, W1u, b1u, W2u, b2u, W1a, b1a, W2a, b2a):
    BLK = 4096
    grid = BATCH // BLK
    espec = pl.BlockSpec((BLK, 2 * EMBED), lambda i: (i, 0))
    ispec = pl.BlockSpec((BLK, 1), lambda i: (i, 0))
    wspec = pl.BlockSpec((EMBED, HID), lambda i: (0, 0))
    w2spec = pl.BlockSpec((HID, HID), lambda i: (0, 0))
    bspec = pl.BlockSpec((1, HID), lambda i: (0, 0))
    return pl.pallas_call(
        _mlp_body,
        grid=(grid,),
        in_specs=[espec, espec, ispec, ispec,
                  wspec, bspec, w2spec, bspec,
                  wspec, bspec, w2spec, bspec],
        out_specs=pl.BlockSpec((BLK,), lambda i: (i,)),
        out_shape=jax.ShapeDtypeStruct((BATCH,), jnp.float32),
    )(gu, ga, uids.reshape(BATCH, 1), aids.reshape(BATCH, 1),
      W1u.T, b1u.reshape(1, HID), W2u.T, b2u.reshape(1, HID),
      W1a.T, b1a.reshape(1, HID), W2a.T, b2a.reshape(1, HID))


def kernel(user_ids, anime_ids, user_table, anime_table,
           W1u, b1u, W2u, b2u, W1a, b1a, W2a, b2a):
    uids = user_ids.astype(jnp.int32)
    aids = anime_ids.astype(jnp.int32)
    apk = _tc_pack_transpose(anime_table.T)
    ga = _sc_gather(apk, aids)
    upk = _tc_pack_transpose(user_table.T)
    gu = _sc_gather(upk, uids)
    return _tc_mlp(gu, ga, uids, aids,
                   W1u, b1u, W2u, b2u, W1a, b1a, W2a, b2a)
